# baseline (speedup 1.0000x reference)
import functools

import jax
import jax.numpy as jnp
from jax import lax
from jax.experimental import pallas as pl
from jax.experimental.pallas import tpu as pltpu
from jax.experimental.pallas import tpu_sc as plsc

T = 16
A = 32
L = 64
N = 32768
K = 1024
ITERS = 2
MIN_STD = 0.05
MAX_STD = 2.0
TEMP = 0.5
RHO = 0.99

D = T * A
B1 = 2048
ROWS = N // D

_INT_MIN = -(2 ** 31)
_POS_HI = 0x7F800000
_NEG_LO = -2139095041


def _sortable_key(s):
    i = lax.bitcast_convert_type(s, jnp.int32)
    return jnp.where(i >= 0, i, jnp.bitwise_not(i ^ jnp.int32(_INT_MIN)))


def _k1_body(eps_ref, s1_ref, s2_ref, z0_ref, dz_ref, wa_ref, wv_ref,
             sc_ref, mean_ref, std_ref):
    mean = jnp.sum(s1_ref[...], axis=0, keepdims=True)
    es2 = jnp.sum(s2_ref[...], axis=0, keepdims=True)
    var = es2 - mean * mean
    std = jnp.clip(jnp.sqrt(jnp.clip(var, 0.0, None)), MIN_STD, MAX_STD)

    @pl.when(pl.program_id(0) == 0)
    def _():
        mean_ref[...] = mean
        std_ref[...] = std

    a = jnp.clip(mean + std * eps_ref[...], -1.0, 1.0)
    z = jnp.broadcast_to(z0_ref[...], (B1, L))
    dz = dz_ref[...]
    val = jnp.zeros((B1, 1), jnp.float32)
    disc = 1.0
    for t in range(T):
        at = a[:, t * A:(t + 1) * A]
        z = jnp.tanh(z * dz + jnp.dot(at, wa_ref[...],
                                      preferred_element_type=jnp.float32))
        val = val + disc * jnp.dot(z, wv_ref[...],
                                   preferred_element_type=jnp.float32)
        disc = disc * RHO
    sc_ref[...] = val


def _scores(eps2d, s1p, s2p, z0r, dzr, wa, wvr):
    grid = N // B1
    return pl.pallas_call(
        _k1_body,
        grid=(grid,),
        in_specs=[
            pl.BlockSpec((B1, D), lambda i: (i, 0)),
            pl.BlockSpec(s1p.shape, lambda i: (0, 0)),
            pl.BlockSpec(s2p.shape, lambda i: (0, 0)),
            pl.BlockSpec((1, L), lambda i: (0, 0)),
            pl.BlockSpec((1, L), lambda i: (0, 0)),
            pl.BlockSpec((A, L), lambda i: (0, 0)),
            pl.BlockSpec((L, 1), lambda i: (0, 0)),
        ],
        out_specs=[
            pl.BlockSpec((B1, 1), lambda i: (i, 0)),
            pl.BlockSpec((1, D), lambda i: (0, 0)),
            pl.BlockSpec((1, D), lambda i: (0, 0)),
        ],
        out_shape=[
            jax.ShapeDtypeStruct((N, 1), jnp.float32),
            jax.ShapeDtypeStruct((1, D), jnp.float32),
            jax.ShapeDtypeStruct((1, D), jnp.float32),
        ],
    )(eps2d, s1p, s2p, z0r, dzr, wa, wvr)


def _k2_body(s_ref, w_ref):
    s = s_ref[...]
    key = _sortable_key(s)
    kf = jnp.float32(K)

    def cnt_ge(t):
        return jnp.sum((key >= t).astype(jnp.float32))

    cnt0 = cnt_ge(jnp.int32(0))
    lo0 = jnp.where(cnt0 >= kf, jnp.int32(0), jnp.int32(_NEG_LO))
    hi0 = jnp.where(cnt0 >= kf, jnp.int32(_POS_HI), jnp.int32(-1))

    def body(_, lh):
        lo, hi = lh
        mid = lo + ((hi - lo + 1) >> 1)
        p = cnt_ge(mid) >= kf
        return (jnp.where(p, mid, lo), jnp.where(p, hi, mid - 1))

    theta, _ = lax.fori_loop(0, 31, body, (lo0, hi0))

    gt = key > theta
    eq = key == theta
    cgt = jnp.sum(gt.astype(jnp.float32))
    needed = kf - cgt
    eqf = eq.astype(jnp.float32)
    li = lax.broadcasted_iota(jnp.int32, (D, D), 0)
    pi = lax.broadcasted_iota(jnp.int32, (D, D), 1)
    upper = (li <= pi).astype(jnp.float32)
    prefix = jnp.dot(eqf, upper, preferred_element_type=jnp.float32)
    tot = prefix[:, D - 1:D]
    ri = lax.broadcasted_iota(jnp.int32, (ROWS, ROWS), 0)
    ci = lax.broadcasted_iota(jnp.int32, (ROWS, ROWS), 1)
    lstrict = (ci < ri).astype(jnp.float32)
    rowoff = jnp.dot(lstrict, tot, preferred_element_type=jnp.float32)
    grank = prefix + rowoff
    sel = gt | (eq & (grank <= needed))

    m = jnp.max(s)
    inv_t = 1.0 / TEMP
    p = jnp.where(sel, jnp.exp(s * inv_t - m * inv_t), 0.0)
    w_ref[...] = p / jnp.sum(p)


def _weights(s2d):
    return pl.pallas_call(
        _k2_body,
        out_shape=jax.ShapeDtypeStruct((ROWS, D), jnp.float32),
    )(s2d)


def _k3_body(eps_ref, w_ref, mean_ref, std_ref, s1_ref, s2_ref):
    a = jnp.clip(mean_ref[...] + std_ref[...] * eps_ref[...], -1.0, 1.0)
    w = w_ref[...]
    dn = (((0,), (0,)), ((), ()))
    wa = lax.dot_general(w, a, dn, preferred_element_type=jnp.float32)
    waa = lax.dot_general(w, a * a, dn, preferred_element_type=jnp.float32)

    @pl.when(pl.program_id(0) == 0)
    def _():
        s1_ref[...] = wa
        s2_ref[...] = waa

    @pl.when(pl.program_id(0) != 0)
    def _():
        s1_ref[...] += wa
        s2_ref[...] += waa


def _elite_update(eps2d, wcol, meanr, stdr):
    grid = N // B1
    return pl.pallas_call(
        _k3_body,
        grid=(grid,),
        in_specs=[
            pl.BlockSpec((B1, D), lambda i: (i, 0)),
            pl.BlockSpec((B1, 1), lambda i: (i, 0)),
            pl.BlockSpec((1, D), lambda i: (0, 0)),
            pl.BlockSpec((1, D), lambda i: (0, 0)),
        ],
        out_specs=[
            pl.BlockSpec((1, D), lambda i: (0, 0)),
            pl.BlockSpec((1, D), lambda i: (0, 0)),
        ],
        out_shape=[
            jax.ShapeDtypeStruct((1, D), jnp.float32),
            jax.ShapeDtypeStruct((1, D), jnp.float32),
        ],
    )(eps2d, wcol, meanr, stdr)


NW = 32
WCHUNK = N // NW
GCH = 32
CAP = K + GCH + 16
NV = D // 16


def _k3sc_body(w_hbm, eps_hbm, mean_hbm, std_hbm, s1p_hbm, s2p_hbm,
               wv, meanv, stdv, idxb, wb, rows, s1, s2, sem):
    cid = lax.axis_index("c")
    sid = lax.axis_index("s")
    wid = sid * 2 + cid
    base = wid * WCHUNK
    pltpu.sync_copy(w_hbm.at[pl.ds(base, WCHUNK)], wv)
    pltpu.sync_copy(mean_hbm, meanv)
    pltpu.sync_copy(std_hbm, stdv)

    zf = jnp.zeros((16,), jnp.float32)
    zi = jnp.zeros((16,), jnp.int32)
    for u in range(NV):
        s1[pl.ds(u * 16, 16)] = zf
        s2[pl.ds(u * 16, 16)] = zf
    iota16 = lax.iota(jnp.int32, 16)

    cnt = jnp.int32(0)
    for j in range(WCHUNK // 16):
        vals = wv[pl.ds(j * 16, 16)]
        msk = vals > 0.0
        ones = msk.astype(jnp.int32)
        pos = cnt + plsc.cumsum(ones) - 1
        gidx = base + j * 16 + iota16
        plsc.store_scatter(idxb, [pos], gidx, mask=msk)
        plsc.store_scatter(wb, [pos], vals, mask=msk)
        cnt = cnt + jnp.sum(ones)

    fullm = iota16 >= 0
    plsc.store_scatter(idxb, [cnt + iota16], zi, mask=fullm)
    plsc.store_scatter(wb, [cnt + iota16], zf, mask=fullm)
    plsc.store_scatter(idxb, [cnt + 16 + iota16], zi, mask=fullm)
    plsc.store_scatter(wb, [cnt + 16 + iota16], zf, mask=fullm)

    mvals = [meanv[pl.ds(u * 16, 16)] for u in range(NV)]
    svals = [stdv[pl.ds(u * 16, 16)] for u in range(NV)]

    nch = (cnt + (GCH - 1)) // GCH

    def ch_body(ci, carry):
        pltpu.async_copy(eps_hbm.at[idxb.at[pl.ds(ci * GCH, GCH)]],
                         rows, sem).wait()

        def r_body(r, carry2):
            wr = wb[pl.ds(ci * GCH + r, 16)][0]
            for u in range(NV):
                e = rows[r, pl.ds(u * 16, 16)]
                a = jnp.clip(mvals[u] + svals[u] * e, -1.0, 1.0)
                wa = wr * a
                plsc.addupdate(s1.at[pl.ds(u * 16, 16)], wa)
                plsc.addupdate(s2.at[pl.ds(u * 16, 16)], wa * a)
            return carry2

        return lax.fori_loop(0, GCH, r_body, carry)

    lax.fori_loop(0, nch, ch_body, jnp.int32(0))

    pltpu.sync_copy(s1, s1p_hbm.at[wid])
    pltpu.sync_copy(s2, s2p_hbm.at[wid])


def _elite_update_sc(wflat, eps2d, meanf, stdf):
    mesh = plsc.VectorSubcoreMesh(core_axis_name="c", subcore_axis_name="s")
    f = functools.partial(
        pl.kernel,
        mesh=mesh,
        compiler_params=pltpu.CompilerParams(needs_layout_passes=False),
        out_type=[
            jax.ShapeDtypeStruct((NW, D), jnp.float32),
            jax.ShapeDtypeStruct((NW, D), jnp.float32),
        ],
        scratch_types=[
            pltpu.VMEM((WCHUNK,), jnp.float32),
            pltpu.VMEM((D,), jnp.float32),
            pltpu.VMEM((D,), jnp.float32),
            pltpu.VMEM((CAP,), jnp.int32),
            pltpu.VMEM((CAP,), jnp.float32),
            pltpu.VMEM((GCH, D), jnp.float32),
            pltpu.VMEM((D,), jnp.float32),
            pltpu.VMEM((D,), jnp.float32),
            pltpu.SemaphoreType.DMA,
        ],
    )(_k3sc_body)
    return f(wflat, eps2d, meanf, stdf)


def _k4_body(s1_ref, out_ref):
    out_ref[...] = jnp.sum(s1_ref[...], axis=0, keepdims=True)


def _reduce_partials(s1p):
    return pl.pallas_call(
        _k4_body,
        out_shape=jax.ShapeDtypeStruct((1, D), jnp.float32),
    )(s1p)


@jax.jit
def kernel(z0, prev_mean, dz, Wa, wv, eps):
    z0r = z0.reshape(1, L)
    dzr = dz.reshape(1, L)
    wvr = wv.reshape(L, 1)
    eps2d = eps.reshape(ITERS, N, D)

    shifted = jnp.zeros_like(prev_mean).at[:-1].set(prev_mean[1:])
    m0 = shifted.reshape(1, D)
    s1p = m0
    s2p = MAX_STD * MAX_STD + m0 * m0

    for it in range(ITERS):
        e = eps2d[it]
        scores, meanr, stdr = _scores(e, s1p, s2p, z0r, dzr, Wa, wvr)
        w2d = _weights(scores.reshape(ROWS, D))
        s1p, s2p = _elite_update_sc(w2d.reshape(N), e,
                                    meanr.reshape(D), stdr.reshape(D))

    mean_final = _reduce_partials(s1p).reshape(T, A)
    return mean_final


# speedup vs baseline: 2.0008x; 2.0008x over previous
import functools

import jax
import jax.numpy as jnp
from jax import lax
from jax.experimental import pallas as pl
from jax.experimental.pallas import tpu as pltpu
from jax.experimental.pallas import tpu_sc as plsc

T = 16
A = 32
L = 64
N = 32768
K = 1024
ITERS = 2
MIN_STD = 0.05
MAX_STD = 2.0
TEMP = 0.5
RHO = 0.99

D = T * A
BN = 2048

_INT_MIN = -(2 ** 31)
_POS_HI = 0x7F800000
_NEG_LO = -2139095041


def _sortable_key(s):
    i = lax.bitcast_convert_type(s, jnp.int32)
    return jnp.where(i >= 0, i, jnp.bitwise_not(i ^ jnp.int32(_INT_MIN)))


def _k1_body(eps_ref, s1_ref, s2_ref, z0_ref, dz_ref, wa_ref, wv_ref,
             sc_ref, mean_ref, std_ref):
    mean = s1_ref[...]
    var = s2_ref[...] - mean * mean
    std = jnp.clip(jnp.sqrt(jnp.clip(var, 0.0, None)), MIN_STD, MAX_STD)

    @pl.when(pl.program_id(0) == 0)
    def _():
        mean_ref[...] = mean
        std_ref[...] = std

    aT = jnp.clip(mean + std * eps_ref[...], -1.0, 1.0)
    zT = jnp.broadcast_to(z0_ref[...], (L, BN))
    dzc = dz_ref[...]
    valT = jnp.zeros((1, BN), jnp.float32)
    disc = 1.0
    dn = (((0,), (0,)), ((), ()))
    for t in range(T):
        atT = aT[t * A:(t + 1) * A, :]
        zT = jnp.tanh(zT * dzc + lax.dot_general(
            wa_ref[...], atT, dn, preferred_element_type=jnp.float32))
        valT = valT + disc * jnp.dot(wv_ref[...], zT,
                                     preferred_element_type=jnp.float32)
        disc = disc * RHO
    i = pl.program_id(0)
    sc_ref[:, pl.ds(pl.multiple_of(i * BN, BN), BN)] = valT


def _scores(epsT, it, s1, s2, z0c, dzc, wa, wvr):
    grid = N // BN
    return pl.pallas_call(
        _k1_body,
        grid=(grid,),
        in_specs=[
            pl.BlockSpec((D, BN), lambda i, _it=it: (_it, i)),
            pl.BlockSpec((D, 1), lambda i: (0, 0)),
            pl.BlockSpec((D, 1), lambda i: (0, 0)),
            pl.BlockSpec((L, 1), lambda i: (0, 0)),
            pl.BlockSpec((L, 1), lambda i: (0, 0)),
            pl.BlockSpec((A, L), lambda i: (0, 0)),
            pl.BlockSpec((1, L), lambda i: (0, 0)),
        ],
        out_specs=[
            pl.BlockSpec((1, N), lambda i: (0, 0)),
            pl.BlockSpec((D, 1), lambda i: (0, 0)),
            pl.BlockSpec((D, 1), lambda i: (0, 0)),
        ],
        out_shape=[
            jax.ShapeDtypeStruct((1, N), jnp.float32),
            jax.ShapeDtypeStruct((D, 1), jnp.float32),
            jax.ShapeDtypeStruct((D, 1), jnp.float32),
        ],
    )(epsT, s1, s2, z0c, dzc, wa, wvr)


def _k2_body(s_ref, w_ref):
    s = s_ref[...]
    key = _sortable_key(s)
    kf = jnp.float32(K)

    def cnt_ge(t):
        return jnp.sum((key >= t).astype(jnp.float32))

    cnt0 = cnt_ge(jnp.int32(0))
    lo0 = jnp.where(cnt0 >= kf, jnp.int32(0), jnp.int32(_NEG_LO))
    hi0 = jnp.where(cnt0 >= kf, jnp.int32(_POS_HI), jnp.int32(-1))

    def vbody(_, lh):
        lo, hi = lh
        mid = lo + ((hi - lo + 1) >> 1)
        p = cnt_ge(mid) >= kf
        return (jnp.where(p, mid, lo), jnp.where(p, hi, mid - 1))

    theta, _ = lax.fori_loop(0, 31, vbody, (lo0, hi0))

    gt = key > theta
    eq = key == theta
    cgt = jnp.sum(gt.astype(jnp.float32))
    needed = kf - cgt
    pos = lax.broadcasted_iota(jnp.int32, (1, N), 1)

    def cnt_le(p):
        return jnp.sum((eq & (pos <= p)).astype(jnp.float32))

    def pbody(_, lh):
        lo, hi = lh
        mid = (lo + hi) >> 1
        sel = cnt_le(mid) >= needed
        return (jnp.where(sel, lo, mid + 1), jnp.where(sel, mid, hi))

    pstar, _ = lax.fori_loop(0, 15, pbody, (jnp.int32(0), jnp.int32(N - 1)))

    sel = gt | (eq & (pos <= pstar))
    m = jnp.max(s)
    inv_t = 1.0 / TEMP
    p = jnp.where(sel, jnp.exp(s * inv_t - m * inv_t), 0.0)
    w_ref[...] = p / jnp.sum(p)


def _weights(scores):
    return pl.pallas_call(
        _k2_body,
        out_shape=jax.ShapeDtypeStruct((1, N), jnp.float32),
    )(scores)


def _k3_body(eps_ref, w_ref, mean_ref, std_ref, s1_ref, s2_ref):
    aT = jnp.clip(mean_ref[...] + std_ref[...] * eps_ref[...], -1.0, 1.0)
    w = w_ref[...]
    dn = (((1,), (1,)), ((), ()))
    s1b = lax.dot_general(aT, w, dn, preferred_element_type=jnp.float32)
    s2b = lax.dot_general(aT * aT, w, dn, preferred_element_type=jnp.float32)

    @pl.when(pl.program_id(0) == 0)
    def _():
        s1_ref[...] = s1b
        s2_ref[...] = s2b

    @pl.when(pl.program_id(0) != 0)
    def _():
        s1_ref[...] += s1b
        s2_ref[...] += s2b


def _elite_update(epsT, it, w, meanc, stdc):
    grid = N // BN
    return pl.pallas_call(
        _k3_body,
        grid=(grid,),
        in_specs=[
            pl.BlockSpec((D, BN), lambda i, _it=it: (_it, i)),
            pl.BlockSpec((1, BN), lambda i: (0, i)),
            pl.BlockSpec((D, 1), lambda i: (0, 0)),
            pl.BlockSpec((D, 1), lambda i: (0, 0)),
        ],
        out_specs=[
            pl.BlockSpec((D, 1), lambda i: (0, 0)),
            pl.BlockSpec((D, 1), lambda i: (0, 0)),
        ],
        out_shape=[
            jax.ShapeDtypeStruct((D, 1), jnp.float32),
            jax.ShapeDtypeStruct((D, 1), jnp.float32),
        ],
    )(epsT, w, meanc, stdc)


@jax.jit
def kernel(z0, prev_mean, dz, Wa, wv, eps):
    epsT = jnp.transpose(eps, (0, 2, 3, 1)).reshape(ITERS * D, N)
    z0c = z0.reshape(L, 1)
    dzc = dz.reshape(L, 1)
    wvr = wv.reshape(1, L)

    shifted = jnp.zeros_like(prev_mean).at[:-1].set(prev_mean[1:])
    m0 = shifted.reshape(D, 1)
    s1 = m0
    s2 = MAX_STD * MAX_STD + m0 * m0

    for it in range(ITERS):
        scores, meanc, stdc = _scores(epsT, it, s1, s2, z0c, dzc, Wa, wvr)
        w = _weights(scores)
        s1, s2 = _elite_update(epsT, it, w, meanc, stdc)

    return s1.reshape(T, A)


# speedup vs baseline: 2.1203x; 1.0598x over previous
import jax
import jax.numpy as jnp
from jax import lax
from jax.experimental import pallas as pl
from jax.experimental.pallas import tpu as pltpu

T = 16
A = 32
L = 64
N = 32768
K = 1024
ITERS = 2
MIN_STD = 0.05
MAX_STD = 2.0
TEMP = 0.5
RHO = 0.99

D = T * A
BN = 2048
GB = N // BN

_INT_MIN = -(2 ** 31)
_POS_HI = 0x7F800000
_NEG_LO = -2139095041


def _selection_weights(s):
    i = lax.bitcast_convert_type(s, jnp.int32)
    key = jnp.where(i >= 0, i, jnp.bitwise_not(i ^ jnp.int32(_INT_MIN)))
    kf = jnp.float32(K)

    def cnt_ge(t):
        return jnp.sum((key >= t).astype(jnp.float32))

    cnt0 = cnt_ge(jnp.int32(0))
    lo0 = jnp.where(cnt0 >= kf, jnp.int32(0), jnp.int32(_NEG_LO))
    hi0 = jnp.where(cnt0 >= kf, jnp.int32(_POS_HI), jnp.int32(-1))

    def vbody(_, lh):
        lo, hi = lh
        mid = lo + ((hi - lo + 1) >> 1)
        p = cnt_ge(mid) >= kf
        return (jnp.where(p, mid, lo), jnp.where(p, hi, mid - 1))

    theta, _ = lax.fori_loop(0, 31, vbody, (lo0, hi0))

    gt = key > theta
    eq = key == theta
    cgt = jnp.sum(gt.astype(jnp.float32))
    needed = kf - cgt
    pos = lax.broadcasted_iota(jnp.int32, (1, N), 1)

    def cnt_le(p):
        return jnp.sum((eq & (pos <= p)).astype(jnp.float32))

    def pbody(_, lh):
        lo, hi = lh
        mid = (lo + hi) >> 1
        ok = cnt_le(mid) >= needed
        return (jnp.where(ok, lo, mid + 1), jnp.where(ok, mid, hi))

    pstar, _ = lax.fori_loop(0, 15, pbody, (jnp.int32(0), jnp.int32(N - 1)))

    sel = gt | (eq & (pos <= pstar))
    m = jnp.max(s)
    inv_t = 1.0 / TEMP
    p = jnp.where(sel, jnp.exp(s * inv_t - m * inv_t), 0.0)
    return p / jnp.sum(p)


def _fused_body(eps_ref, s10_ref, s20_ref, z0_ref, dz_ref, wa_ref, wv_ref,
                mean_out_ref, sc_s, w_s, s1_s, s2_s, mean_s, std_s):
    it = pl.program_id(0)
    ph = pl.program_id(1)
    i = pl.program_id(2)

    @pl.when((ph == 0) & (i == 0))
    def _():
        first = it == 0
        mean = jnp.where(first, s10_ref[...], s1_s[...])
        es2 = jnp.where(first, s20_ref[...], s2_s[...])
        var = es2 - mean * mean
        std = jnp.clip(jnp.sqrt(jnp.clip(var, 0.0, None)), MIN_STD, MAX_STD)
        mean_s[...] = mean
        std_s[...] = std
        zd = jnp.zeros((D, 1), jnp.float32)
        s1_s[...] = zd
        s2_s[...] = zd

    lane0 = pl.ds(pl.multiple_of(i * BN, BN), BN)

    @pl.when(ph == 0)
    def _():
        aT = jnp.clip(mean_s[...] + std_s[...] * eps_ref[...], -1.0, 1.0)
        zT = jnp.broadcast_to(z0_ref[...], (L, BN))
        dzc = dz_ref[...]
        valT = jnp.zeros((1, BN), jnp.float32)
        disc = 1.0
        dn = (((0,), (0,)), ((), ()))
        for t in range(T):
            atT = aT[t * A:(t + 1) * A, :]
            zT = jnp.tanh(zT * dzc + lax.dot_general(
                wa_ref[...], atT, dn, preferred_element_type=jnp.float32))
            valT = valT + disc * jnp.dot(wv_ref[...], zT,
                                         preferred_element_type=jnp.float32)
            disc = disc * RHO
        sc_s[:, lane0] = valT

    @pl.when((ph == 0) & (i == GB - 1))
    def _():
        w_s[...] = _selection_weights(sc_s[...])

    @pl.when(ph == 1)
    def _():
        aT = jnp.clip(mean_s[...] + std_s[...] * eps_ref[...], -1.0, 1.0)
        wblk = w_s[:, lane0]
        dnl = (((1,), (1,)), ((), ()))
        s1_s[...] += lax.dot_general(aT, wblk, dnl,
                                     preferred_element_type=jnp.float32)
        s2_s[...] += lax.dot_general(aT * aT, wblk, dnl,
                                     preferred_element_type=jnp.float32)

    @pl.when((it == ITERS - 1) & (ph == 1) & (i == GB - 1))
    def _():
        mean_out_ref[...] = s1_s[...]


def _planner(epsT, s10, s20, z0c, dzc, wa, wvr):
    return pl.pallas_call(
        _fused_body,
        grid=(ITERS, 2, GB),
        in_specs=[
            pl.BlockSpec((D, BN), lambda it, ph, i: (it, i)),
            pl.BlockSpec((D, 1), lambda it, ph, i: (0, 0)),
            pl.BlockSpec((D, 1), lambda it, ph, i: (0, 0)),
            pl.BlockSpec((L, 1), lambda it, ph, i: (0, 0)),
            pl.BlockSpec((L, 1), lambda it, ph, i: (0, 0)),
            pl.BlockSpec((A, L), lambda it, ph, i: (0, 0)),
            pl.BlockSpec((1, L), lambda it, ph, i: (0, 0)),
        ],
        out_specs=pl.BlockSpec((D, 1), lambda it, ph, i: (0, 0)),
        out_shape=jax.ShapeDtypeStruct((D, 1), jnp.float32),
        scratch_shapes=[
            pltpu.VMEM((1, N), jnp.float32),
            pltpu.VMEM((1, N), jnp.float32),
            pltpu.VMEM((D, 1), jnp.float32),
            pltpu.VMEM((D, 1), jnp.float32),
            pltpu.VMEM((D, 1), jnp.float32),
            pltpu.VMEM((D, 1), jnp.float32),
        ],
    )(epsT, s10, s20, z0c, dzc, wa, wvr)


@jax.jit
def kernel(z0, prev_mean, dz, Wa, wv, eps):
    epsT = jnp.transpose(eps, (0, 2, 3, 1)).reshape(ITERS * D, N)
    z0c = z0.reshape(L, 1)
    dzc = dz.reshape(L, 1)
    wvr = wv.reshape(1, L)

    shifted = jnp.zeros_like(prev_mean).at[:-1].set(prev_mean[1:])
    m0 = shifted.reshape(D, 1)
    s10 = m0
    s20 = MAX_STD * MAX_STD + m0 * m0

    mean_final = _planner(epsT, s10, s20, z0c, dzc, Wa, wvr)
    return mean_final.reshape(T, A)
